# baseline (speedup 1.0000x reference)
import functools

import jax
import jax.numpy as jnp
from jax.experimental import pallas as pl

_SELU_SCALE = 1.0507009873554804934193349852946
_SELU_ALPHA = 1.6732632423543772848170429916717


def _body(a_ref, x_ref, w_ref, o_ref):
    t = pl.program_id(1)
    a = a_ref[0]
    x = x_ref[t]
    h = jnp.dot(a, x, preferred_element_type=jnp.float32)
    h = jnp.dot(h, w_ref[...], preferred_element_type=jnp.float32)
    h = _SELU_SCALE * jnp.where(h > 0, h, _SELU_ALPHA * (jnp.exp(h) - 1.0))
    o_ref[:, t, :] = h


@functools.partial(jax.jit, static_argnames=("block_n",))
def _run(Feature_tensor, adj, W, block_n=256):
    n_time, n_node, d = Feature_tensor.shape
    grid = (n_node // block_n, n_time)
    return pl.pallas_call(
        _body,
        grid=grid,
        in_specs=[
            pl.BlockSpec((1, block_n, n_node), lambda i, t: (t, i, 0)),
            pl.BlockSpec((n_time, n_node, d), lambda i, t: (0, 0, 0)),
            pl.BlockSpec((d, d), lambda i, t: (0, 0)),
        ],
        out_specs=pl.BlockSpec((block_n, n_time, d), lambda i, t: (i, 0, 0)),
        out_shape=jax.ShapeDtypeStruct((n_node, n_time, d), jnp.float32),
    )(adj, Feature_tensor, W)


def kernel(Feature_tensor, adj, W):
    return _run(Feature_tensor, adj, W)
